# Initial kernel scaffold; baseline (speedup 1.0000x reference)
import jax
import jax.numpy as jnp
from jax.experimental import pallas as pl


def kernel(x, W1, b1, W2, b2, We, be):
    raise NotImplementedError("write your pallas kernel here")


# speedup vs baseline: 5.0475x; 5.0475x over previous
import functools

import jax
import jax.numpy as jnp
from jax.experimental import pallas as pl
from jax.experimental.pallas import tpu as pltpu

N, D, F, E, H = 8192, 768, 768, 8, 64
BT = 512


def _moe_block(x_ref, w1_ref, b1_ref, w2_ref, b2_ref, we_ref, be_ref,
               out_ref, gw_ref, idx_ref):
    x = x_ref[...]
    h = jnp.maximum(
        jnp.dot(x, w1_ref[...], preferred_element_type=jnp.float32)
        + b1_ref[...], 0.0)
    scores = jnp.dot(h, w2_ref[...], preferred_element_type=jnp.float32) \
        + b2_ref[...]
    m = jnp.max(scores, axis=1, keepdims=True)
    ex = jnp.exp(scores - m)
    gw = ex / jnp.sum(ex, axis=1, keepdims=True)
    gw_ref[...] = gw

    lanes = jax.lax.broadcasted_iota(jnp.int32, (BT, E), 1)
    m1 = jnp.max(gw, axis=1, keepdims=True)
    a1 = jnp.min(jnp.where(gw == m1, lanes, E), axis=1, keepdims=True)
    rest = gw - jnp.where(lanes == a1, jnp.inf, 0.0)
    m2 = jnp.max(rest, axis=1, keepdims=True)
    a2 = jnp.min(jnp.where(rest == m2, lanes, E), axis=1, keepdims=True)
    idx_ref[...] = jnp.concatenate([a1, a2], axis=1)

    acc = jnp.zeros((BT, F), jnp.float32)
    for e in range(E):
        sel = (a1[:, 0] == e) | (a2[:, 0] == e)
        w_e = jnp.where(sel, gw[:, e], 0.0)
        y = jnp.dot(x, we_ref[e], preferred_element_type=jnp.float32) \
            + be_ref[e]
        acc = acc + y * w_e[:, None]
    out_ref[...] = acc


@jax.jit
def kernel(x, W1, b1, W2, b2, We, be):
    grid = (N // BT,)
    out, gw, idx = pl.pallas_call(
        _moe_block,
        grid=grid,
        in_specs=[
            pl.BlockSpec((BT, D), lambda i: (i, 0)),
            pl.BlockSpec((D, H), lambda i: (0, 0)),
            pl.BlockSpec((H,), lambda i: (0,)),
            pl.BlockSpec((H, E), lambda i: (0, 0)),
            pl.BlockSpec((E,), lambda i: (0,)),
            pl.BlockSpec((E, D, F), lambda i: (0, 0, 0)),
            pl.BlockSpec((E, F), lambda i: (0, 0)),
        ],
        out_specs=[
            pl.BlockSpec((BT, F), lambda i: (i, 0)),
            pl.BlockSpec((BT, E), lambda i: (i, 0)),
            pl.BlockSpec((BT, 2), lambda i: (i, 0)),
        ],
        out_shape=[
            jax.ShapeDtypeStruct((N, F), jnp.float32),
            jax.ShapeDtypeStruct((N, E), jnp.float32),
            jax.ShapeDtypeStruct((N, 2), jnp.int32),
        ],
    )(x, W1, b1, W2, b2, We, be)
    return (out, gw, idx)
